# baseline (speedup 1.0000x reference)
import functools

import jax
import jax.numpy as jnp
from jax import lax
from jax.experimental import pallas as pl
from jax.experimental.pallas import tpu as pltpu
from jax.experimental.pallas import tpu_sc as plsc

_EPB = 2
_LANES = 16


def _sc_dispatch(ids_flat, tw_flat, m, e_total, topk):
    info = plsc.get_sparse_core_info()
    nc, ns = info.num_cores, info.num_subcores
    nw = nc * ns
    tok_pw = m // nw
    slots_pw = tok_pw * topk
    out_pw = tok_pw * e_total
    mesh = plsc.VectorSubcoreMesh(core_axis_name="c", subcore_axis_name="s")

    @functools.partial(
        pl.kernel, mesh=mesh,
        out_type=jax.ShapeDtypeStruct((m * e_total,), jnp.float32),
        scratch_types=[
            pltpu.VMEM((slots_pw,), jnp.int32),
            pltpu.VMEM((slots_pw,), jnp.float32),
            pltpu.VMEM((slots_pw,), jnp.int32),
            pltpu.VMEM((out_pw,), jnp.float32),
            pltpu.VMEM_SHARED((ns * out_pw,), jnp.float32),
        ],
    )
    def k(ids_hbm, tw_hbm, out_hbm, ids_v, tw_v, idx_v, zero_v, shared):
        s_idx = lax.axis_index("s")
        c_idx = lax.axis_index("c")
        wid = s_idx * nc + c_idx
        pltpu.sync_copy(ids_hbm.at[pl.ds(wid * slots_pw, slots_pw)], ids_v)
        pltpu.sync_copy(tw_hbm.at[pl.ds(wid * slots_pw, slots_pw)], tw_v)
        iota = lax.iota(jnp.int32, _LANES)
        for b in range(slots_pw // _LANES):
            eid = ids_v[pl.ds(b * _LANES, _LANES)]
            lslot = iota + b * _LANES
            ltok = lax.shift_right_logical(lslot, topk.bit_length() - 1)
            idx = (s_idx * tok_pw + ltok) * e_total + eid
            idx_v[pl.ds(b * _LANES, _LANES)] = idx
        for b in range(out_pw // _LANES):
            zero_v[pl.ds(b * _LANES, _LANES)] = jnp.zeros((_LANES,),
                                                          jnp.float32)
        pltpu.sync_copy(zero_v, shared.at[pl.ds(s_idx * out_pw, out_pw)])
        pltpu.sync_copy(tw_v, shared.at[idx_v], add=True)
        pltpu.sync_copy(shared.at[pl.ds(s_idx * out_pw, out_pw)],
                        out_hbm.at[pl.ds(wid * out_pw, out_pw)])

    return k(ids_flat, tw_flat)


def _moe_body(x_ref, w1g_ref, w1u_ref, w2a_ref, w2b_ref, wpe_ref, out_ref):
    g = pl.program_id(0)
    x = x_ref[...]
    dn = (((1,), (1,)), ((), ()))
    kh = w2a_ref.shape[2]
    for i in range(_EPB):
        gate = jax.lax.dot_general(x, w1g_ref[i, 0], dn,
                                   preferred_element_type=jnp.float32)
        up = jax.lax.dot_general(x, w1u_ref[i, 0], dn,
                                 preferred_element_type=jnp.float32)
        act = gate * jax.nn.sigmoid(gate) * up
        ya = jax.lax.dot_general(act, w2a_ref[i, 0], dn,
                                 preferred_element_type=jnp.float32)
        yb = jax.lax.dot_general(act, w2b_ref[i, 0], dn,
                                 preferred_element_type=jnp.float32)
        wpe = wpe_ref[i]
        if i == 0:
            @pl.when(g == 0)
            def _init():
                out_ref[:, :kh] = wpe * ya
                out_ref[:, kh:] = wpe * yb

            @pl.when(g > 0)
            def _acc():
                out_ref[:, :kh] += wpe * ya
                out_ref[:, kh:] += wpe * yb
        else:
            out_ref[:, :kh] += wpe * ya
            out_ref[:, kh:] += wpe * yb


def kernel(hidden_states, w1, w2, topk_weights, topk_ids):
    m, k = hidden_states.shape
    e_total, two_n, _ = w1.shape
    n = w2.shape[2]
    topk = topk_ids.shape[1]
    kh = k // 2
    wpe_flat = _sc_dispatch(topk_ids.reshape(-1), topk_weights.reshape(-1),
                            m, e_total, topk)
    wpe = wpe_flat.reshape(m, e_total).T.reshape(e_total, m, 1)
    w1r = w1.reshape(e_total, 2, n, k)
    w2r = w2.reshape(e_total, 2, kh, n)
    return pl.pallas_call(
        _moe_body,
        grid=(e_total // _EPB,),
        in_specs=[
            pl.BlockSpec((m, k), lambda g: (0, 0)),
            pl.BlockSpec((_EPB, 1, n, k), lambda g: (g, 0, 0, 0)),
            pl.BlockSpec((_EPB, 1, n, k), lambda g: (g, 1, 0, 0)),
            pl.BlockSpec((_EPB, 1, kh, n), lambda g: (g, 0, 0, 0)),
            pl.BlockSpec((_EPB, 1, kh, n), lambda g: (g, 1, 0, 0)),
            pl.BlockSpec((_EPB, m, 1), lambda g: (g, 0, 0)),
        ],
        out_specs=pl.BlockSpec((m, k), lambda g: (0, 0)),
        out_shape=jax.ShapeDtypeStruct((m, k), jnp.float32),
        compiler_params=pltpu.CompilerParams(
            dimension_semantics=("arbitrary",)),
    )(hidden_states, w1r, w1r, w2r, w2r, wpe)


# speedup vs baseline: 1.1793x; 1.1793x over previous
import jax
import jax.numpy as jnp
from jax.experimental import pallas as pl
from jax.experimental.pallas import tpu as pltpu

_EPB = 2


def _moe_body(x_ref, w1g_ref, w1u_ref, w2a_ref, w2b_ref, tw_ref, ids_ref,
              out_ref):
    g = pl.program_id(0)
    x = x_ref[...]
    dn = (((1,), (1,)), ((), ()))
    kh = w2a_ref.shape[2]
    for i in range(_EPB):
        e = g * _EPB + i
        gate = jax.lax.dot_general(x, w1g_ref[i, 0], dn,
                                   preferred_element_type=jnp.float32)
        up = jax.lax.dot_general(x, w1u_ref[i, 0], dn,
                                 preferred_element_type=jnp.float32)
        act = gate * jax.nn.sigmoid(gate) * up
        ya = jax.lax.dot_general(act, w2a_ref[i, 0], dn,
                                 preferred_element_type=jnp.float32)
        yb = jax.lax.dot_general(act, w2b_ref[i, 0], dn,
                                 preferred_element_type=jnp.float32)
        sel = (ids_ref[...] == e).astype(jnp.float32)
        wpe = jnp.sum(tw_ref[...] * sel, axis=1, keepdims=True)
        if i == 0:
            @pl.when(g == 0)
            def _init():
                out_ref[:, :kh] = wpe * ya
                out_ref[:, kh:] = wpe * yb

            @pl.when(g > 0)
            def _acc():
                out_ref[:, :kh] += wpe * ya
                out_ref[:, kh:] += wpe * yb
        else:
            out_ref[:, :kh] += wpe * ya
            out_ref[:, kh:] += wpe * yb


def kernel(hidden_states, w1, w2, topk_weights, topk_ids):
    m, k = hidden_states.shape
    e_total, two_n, _ = w1.shape
    n = w2.shape[2]
    topk = topk_ids.shape[1]
    kh = k // 2
    w1r = w1.reshape(e_total, 2, n, k)
    w2r = w2.reshape(e_total, 2, kh, n)
    return pl.pallas_call(
        _moe_body,
        grid=(e_total // _EPB,),
        in_specs=[
            pl.BlockSpec((m, k), lambda g: (0, 0)),
            pl.BlockSpec((_EPB, 1, n, k), lambda g: (g, 0, 0, 0)),
            pl.BlockSpec((_EPB, 1, n, k), lambda g: (g, 1, 0, 0)),
            pl.BlockSpec((_EPB, 1, kh, n), lambda g: (g, 0, 0, 0)),
            pl.BlockSpec((_EPB, 1, kh, n), lambda g: (g, 1, 0, 0)),
            pl.BlockSpec((m, topk), lambda g: (0, 0)),
            pl.BlockSpec((m, topk), lambda g: (0, 0)),
        ],
        out_specs=pl.BlockSpec((m, k), lambda g: (0, 0)),
        out_shape=jax.ShapeDtypeStruct((m, k), jnp.float32),
        compiler_params=pltpu.CompilerParams(
            dimension_semantics=("arbitrary",)),
    )(hidden_states, w1r, w1r, w2r, w2r, topk_weights, topk_ids)


# speedup vs baseline: 1.2001x; 1.0176x over previous
import jax
import jax.numpy as jnp
from jax.experimental import pallas as pl
from jax.experimental.pallas import tpu as pltpu

_EPB = 2


def _moe_body(x_ref, w1a_ref, w1b_ref, w1c_ref, w1d_ref, w2a_ref, w2b_ref,
              tw_ref, ids_ref, out_ref):
    g = pl.program_id(0)
    x = x_ref[...]
    dn = (((1,), (1,)), ((), ()))
    kh = w2a_ref.shape[2]
    for i in range(_EPB):
        e = g * _EPB + i
        g1 = jax.lax.dot_general(x, w1a_ref[i, 0], dn,
                                 preferred_element_type=jnp.float32)
        g2 = jax.lax.dot_general(x, w1b_ref[i, 0], dn,
                                 preferred_element_type=jnp.float32)
        u1 = jax.lax.dot_general(x, w1c_ref[i, 0], dn,
                                 preferred_element_type=jnp.float32)
        u2 = jax.lax.dot_general(x, w1d_ref[i, 0], dn,
                                 preferred_element_type=jnp.float32)
        act = jnp.concatenate(
            [g1 * jax.nn.sigmoid(g1) * u1, g2 * jax.nn.sigmoid(g2) * u2],
            axis=1)
        ya = jax.lax.dot_general(act, w2a_ref[i, 0], dn,
                                 preferred_element_type=jnp.float32)
        yb = jax.lax.dot_general(act, w2b_ref[i, 0], dn,
                                 preferred_element_type=jnp.float32)
        sel = (ids_ref[...] == e).astype(jnp.float32)
        wpe = jnp.sum(tw_ref[...] * sel, axis=1, keepdims=True)
        if i == 0:
            @pl.when(g == 0)
            def _init():
                out_ref[:, :kh] = wpe * ya
                out_ref[:, kh:] = wpe * yb

            @pl.when(g > 0)
            def _acc():
                out_ref[:, :kh] += wpe * ya
                out_ref[:, kh:] += wpe * yb
        else:
            out_ref[:, :kh] += wpe * ya
            out_ref[:, kh:] += wpe * yb


def kernel(hidden_states, w1, w2, topk_weights, topk_ids):
    m, k = hidden_states.shape
    e_total, two_n, _ = w1.shape
    n = w2.shape[2]
    topk = topk_ids.shape[1]
    nq = two_n // 4
    kh = k // 2
    w1r = w1.reshape(e_total, 4, nq, k)
    w2r = w2.reshape(e_total, 2, kh, n)

    def w1spec(q):
        return pl.BlockSpec((_EPB, 1, nq, k), lambda g, q=q: (g, q, 0, 0))

    def w2spec(q):
        return pl.BlockSpec((_EPB, 1, kh, n), lambda g, q=q: (g, q, 0, 0))

    return pl.pallas_call(
        _moe_body,
        grid=(e_total // _EPB,),
        in_specs=[
            pl.BlockSpec((m, k), lambda g: (0, 0)),
            w1spec(0), w1spec(1), w1spec(2), w1spec(3),
            w2spec(0), w2spec(1),
            pl.BlockSpec((m, topk), lambda g: (0, 0)),
            pl.BlockSpec((m, topk), lambda g: (0, 0)),
        ],
        out_specs=pl.BlockSpec((m, k), lambda g: (0, 0)),
        out_shape=jax.ShapeDtypeStruct((m, k), jnp.float32),
        compiler_params=pltpu.CompilerParams(
            dimension_semantics=("arbitrary",)),
    )(hidden_states, w1r, w1r, w1r, w1r, w2r, w2r, topk_weights, topk_ids)
